# Initial kernel scaffold; baseline (speedup 1.0000x reference)
import jax
import jax.numpy as jnp
from jax.experimental import pallas as pl


def kernel(x):
    raise NotImplementedError("write your pallas kernel here")


# speedup vs baseline: 40.8067x; 40.8067x over previous
import jax
import jax.numpy as jnp
from jax.experimental import pallas as pl

_ROWS_PER_BLOCK = 16


def _decomp_body(x_ref, season_ref, resid_ref):
    i = pl.program_id(0)
    blk = x_ref[...]

    @pl.when(i == 0)
    def _first_block():
        row = jax.lax.broadcasted_iota(jnp.int32, blk.shape, 0)
        is_row0 = row == 0
        season_ref[...] = jnp.where(is_row0, 0.0, blk)
        resid_ref[...] = jnp.where(is_row0, blk, 0.0)

    @pl.when(i != 0)
    def _rest():
        season_ref[...] = blk
        resid_ref[...] = jnp.zeros_like(blk)


def kernel(x):
    n_rows, n_cols = x.shape
    grid = (n_rows // _ROWS_PER_BLOCK,)
    block = pl.BlockSpec((_ROWS_PER_BLOCK, n_cols), lambda i: (i, 0))
    out_shape = jax.ShapeDtypeStruct(x.shape, x.dtype)
    season, resid = pl.pallas_call(
        _decomp_body,
        grid=grid,
        in_specs=[block],
        out_specs=[block, block],
        out_shape=[out_shape, out_shape],
    )(x)
    return (resid, season)
